# Initial kernel scaffold; baseline (speedup 1.0000x reference)
import jax
import jax.numpy as jnp
from jax.experimental import pallas as pl


def kernel(x, edge_index, adj_values, W, b):
    raise NotImplementedError("write your pallas kernel here")


# speedup vs baseline: 4.5505x; 4.5505x over previous
import functools

import jax
import jax.numpy as jnp
from jax import lax
from jax.experimental import pallas as pl
from jax.experimental.pallas import tpu as pltpu
from jax.experimental.pallas import tpu_sc as plsc

N = 10000
E = 320000
D = 128

NC = 2
NS = 16
NW = NC * NS

EPT = E // NW
KB = 80
NB = EPT // KB
NP = 10240
RPS = NP // NS
ZR = 128

_mesh = plsc.VectorSubcoreMesh(core_axis_name="c", subcore_axis_name="s")


@functools.partial(
    pl.kernel,
    mesh=_mesh,
    out_type=jax.ShapeDtypeStruct((NC, NP, D), jnp.float32),
    scratch_types=[
        pltpu.VMEM((KB,), jnp.int32),
        pltpu.VMEM((KB,), jnp.int32),
        pltpu.VMEM((KB,), jnp.float32),
        pltpu.VMEM((KB, D), jnp.float32),
        pltpu.VMEM((ZR, D), jnp.float32),
        pltpu.VMEM_SHARED((NP, D), jnp.float32),
        pltpu.SemaphoreType.DMA,
    ],
)
def _sc_agg(x_hbm, src_hbm, dst_hbm, adj_hbm, out_hbm,
            src_v, dst_v, adj_v, rows_v, stage_v, acc_sh, sem):
    c = lax.axis_index("c")
    s = lax.axis_index("s")
    wid = s * NC + c

    zv = jnp.zeros((16,), jnp.float32)

    def zrow(i, carry):
        for k in range(D // 16):
            stage_v[i, pl.ds(k * 16, 16)] = zv
        return carry

    lax.fori_loop(0, ZR, zrow, 0)
    for q in range(RPS // ZR):
        pltpu.sync_copy(stage_v, acc_sh.at[pl.ds(s * RPS + q * ZR, ZR)])
    plsc.subcore_barrier()

    def body(t, carry):
        base = wid * EPT + t * KB
        pltpu.sync_copy(src_hbm.at[pl.ds(base, KB)], src_v)
        pltpu.sync_copy(dst_hbm.at[pl.ds(base, KB)], dst_v)
        pltpu.sync_copy(adj_hbm.at[pl.ds(base, KB)], adj_v)
        pltpu.async_copy(x_hbm.at[src_v], rows_v, sem).wait()

        def sbody(g, inner):
            av = adj_v[pl.ds(g * 16, 16)]
            for e in range(16):
                a = av[e]
                j = g * 16 + e
                for k in range(D // 16):
                    sl = pl.ds(k * 16, 16)
                    rows_v[j, sl] = rows_v[j, sl] * a
            return inner

        lax.fori_loop(0, KB // 16, sbody, 0)
        pltpu.sync_copy(rows_v, acc_sh.at[dst_v], add=True)
        return carry

    lax.fori_loop(0, NB, body, 0)
    plsc.subcore_barrier()

    for q in range(RPS // ZR):
        r0 = s * RPS + q * ZR
        pltpu.sync_copy(acc_sh.at[pl.ds(r0, ZR)], stage_v)
        pltpu.sync_copy(stage_v, out_hbm.at[c, pl.ds(r0, ZR)])


RB = 2000


def _mm_body(p_ref, w_ref, b_ref, o_ref):
    acc = p_ref[0] + p_ref[1]
    o_ref[...] = (
        jnp.dot(acc, w_ref[...], preferred_element_type=jnp.float32)
        + b_ref[...]
    )


_mm = pl.pallas_call(
    _mm_body,
    grid=(N // RB,),
    in_specs=[
        pl.BlockSpec((NC, RB, D), lambda i: (0, i, 0)),
        pl.BlockSpec((D, D), lambda i: (0, 0)),
        pl.BlockSpec((1, D), lambda i: (0, 0)),
    ],
    out_specs=pl.BlockSpec((RB, D), lambda i: (i, 0)),
    out_shape=jax.ShapeDtypeStruct((N, D), jnp.float32),
)


def kernel(x, edge_index, adj_values, W, b):
    ei = edge_index.astype(jnp.int32)
    dst = ei[0]
    src = ei[1]
    partial = _sc_agg(x, src, dst, adj_values)
    return _mm(partial, W.T, b.reshape(1, D))


# speedup vs baseline: 9.6580x; 2.1224x over previous
import functools

import jax
import jax.numpy as jnp
from jax import lax
from jax.experimental import pallas as pl
from jax.experimental.pallas import tpu as pltpu
from jax.experimental.pallas import tpu_sc as plsc

N = 10000
E = 320000
D = 128

NC = 2
NS = 16
NW = NC * NS

EPT = E // NW
KB = 80
NB = EPT // KB
CH = 5
NBC = NB // CH
NP = 10240
RPS = NP // NS

_mesh = plsc.VectorSubcoreMesh(core_axis_name="c", subcore_axis_name="s")


@functools.partial(
    pl.kernel,
    mesh=_mesh,
    out_type=jax.ShapeDtypeStruct((NC, NP, D), jnp.float32),
    scratch_types=[
        pltpu.VMEM((NBC, KB), jnp.int32),
        pltpu.VMEM((NBC, KB), jnp.int32),
        pltpu.VMEM((NBC, KB), jnp.float32),
        pltpu.VMEM((KB, D), jnp.float32),
        pltpu.VMEM((KB, D), jnp.float32),
        pltpu.VMEM_SHARED((NP, D), jnp.float32),
        pltpu.SemaphoreType.DMA,
        pltpu.SemaphoreType.DMA,
    ],
)
def _sc_agg(x_hbm, src_hbm, dst_hbm, adj_hbm, out_hbm,
            srcb, dstb, adjb, rows0, rows1, acc_sh, sem0, sem1):
    c = lax.axis_index("c")
    s = lax.axis_index("s")
    wid = s * NC + c

    zv = jnp.zeros((16,), jnp.float32)

    def zrow(i, carry):
        for k in range(D // 16):
            rows0[i, pl.ds(k * 16, 16)] = zv
        return carry

    lax.fori_loop(0, KB, zrow, 0)
    for q in range(RPS // KB):
        pltpu.sync_copy(rows0, acc_sh.at[pl.ds(s * RPS + q * KB, KB)])
    plsc.subcore_barrier()

    def start_gather(t, rows, sem):
        pltpu.async_copy(x_hbm.at[srcb.at[t]], rows, sem)

    def wait_gather(rows, sem):
        pltpu.make_async_copy(x_hbm.at[srcb.at[0]], rows, sem).wait()

    def process(t, rows):
        def sbody(g, inner):
            av = adjb[t, pl.ds(g * 16, 16)]
            for e in range(16):
                a = av[e]
                j = g * 16 + e
                for k in range(D // 16):
                    sl = pl.ds(k * 16, 16)
                    rows[j, sl] = rows[j, sl] * a
            return inner

        lax.fori_loop(0, KB // 16, sbody, 0)
        pltpu.sync_copy(rows, acc_sh.at[dstb.at[t]], add=True)

    def chunk_body(ch, carry):
        pltpu.sync_copy(src_hbm.at[wid, ch], srcb)
        pltpu.sync_copy(dst_hbm.at[wid, ch], dstb)
        pltpu.sync_copy(adj_hbm.at[wid, ch], adjb)
        start_gather(0, rows0, sem0)

        def body(i, c2):
            t0 = 2 * i
            wait_gather(rows0, sem0)
            start_gather(t0 + 1, rows1, sem1)
            process(t0, rows0)
            wait_gather(rows1, sem1)
            start_gather(t0 + 2, rows0, sem0)
            process(t0 + 1, rows1)
            return c2

        lax.fori_loop(0, (NBC - 1) // 2, body, 0)
        wait_gather(rows0, sem0)
        process(NBC - 1, rows0)
        return carry

    lax.fori_loop(0, CH, chunk_body, 0)
    plsc.subcore_barrier()

    for q in range(RPS // KB):
        r0 = s * RPS + q * KB
        pltpu.sync_copy(acc_sh.at[pl.ds(r0, KB)], rows0)
        pltpu.sync_copy(rows0, out_hbm.at[c, pl.ds(r0, KB)])


RB = 2000


def _mm_body(p_ref, w_ref, b_ref, o_ref):
    acc = p_ref[0] + p_ref[1]
    o_ref[...] = (
        jnp.dot(acc, w_ref[...], preferred_element_type=jnp.float32)
        + b_ref[...]
    )


_mm = pl.pallas_call(
    _mm_body,
    grid=(N // RB,),
    in_specs=[
        pl.BlockSpec((NC, RB, D), lambda i: (0, i, 0)),
        pl.BlockSpec((D, D), lambda i: (0, 0)),
        pl.BlockSpec((1, D), lambda i: (0, 0)),
    ],
    out_specs=pl.BlockSpec((RB, D), lambda i: (i, 0)),
    out_shape=jax.ShapeDtypeStruct((N, D), jnp.float32),
)


def kernel(x, edge_index, adj_values, W, b):
    ei = edge_index.astype(jnp.int32)
    dst = ei[0].reshape(NW, CH, NBC, KB)
    src = ei[1].reshape(NW, CH, NBC, KB)
    adj = adj_values.reshape(NW, CH, NBC, KB)
    partial = _sc_agg(x, src, dst, adj)
    return _mm(partial, W.T, b.reshape(1, D))
